# baseline (speedup 1.0000x reference)
import functools
import math

import jax
import jax.numpy as jnp
from jax import lax
from jax.experimental import pallas as pl
from jax.experimental.pallas import tpu as pltpu
from jax.experimental.pallas import tpu_sc as plsc

_SMOOTHING = 0.1
_CONFIDENCE = 1.0 - _SMOOTHING

_ROW_BLOCK = 256


def _tc_rowsum_body(x1_ref, x2_ref, m1_ref, m2_ref, out_ref):
    rs1 = jnp.sum(x1_ref[...], axis=1)
    rs2 = jnp.sum(x2_ref[...], axis=1)
    m1 = m1_ref[...]
    m2 = m2_ref[...]
    out_ref[0, 0, 0] = jnp.sum(rs1 * m1) + jnp.sum(rs2 * m2)
    out_ref[0, 0, 1] = jnp.sum(m1) + jnp.sum(m2)


def _tc_masked_rowsum(x, m):
    n, v = x.shape
    grid = n // (2 * _ROW_BLOCK)
    half = grid
    return pl.pallas_call(
        _tc_rowsum_body,
        grid=(grid,),
        in_specs=[
            pl.BlockSpec((_ROW_BLOCK, v), lambda i: (i, 0)),
            pl.BlockSpec((_ROW_BLOCK, v), lambda i: (i + half, 0)),
            pl.BlockSpec((_ROW_BLOCK,), lambda i: (i,)),
            pl.BlockSpec((_ROW_BLOCK,), lambda i: (i + half,)),
        ],
        out_specs=pl.BlockSpec(
            (1, 1, 2), lambda i: (i, 0, 0), memory_space=pltpu.SMEM),
        out_shape=jax.ShapeDtypeStruct((grid, 1, 2), jnp.float32),
        compiler_params=pltpu.CompilerParams(
            dimension_semantics=("parallel",)),
    )(x, x, m, m)


def _sc_masked_gather_partials(x_flat, flat_idx, m):
    n = flat_idx.shape[0]
    info = plsc.get_sparse_core_info()
    num_cores, num_subcores, num_lanes = (
        info.num_cores, info.num_subcores, info.num_lanes)
    nw = num_cores * num_subcores
    bpw = n // nw
    chunk = 128
    nchunks = bpw // chunk
    mesh = plsc.VectorSubcoreMesh(core_axis_name="c", subcore_axis_name="s")

    @functools.partial(
        pl.kernel,
        mesh=mesh,
        out_type=jax.ShapeDtypeStruct((nw, num_lanes), jnp.float32),
        scratch_types=[
            pltpu.VMEM((chunk,), jnp.int32),
            pltpu.VMEM((chunk,), jnp.float32),
            pltpu.VMEM((chunk,), jnp.float32),
            pltpu.VMEM((num_lanes,), jnp.float32),
            pltpu.SemaphoreType.DMA,
        ],
    )
    def k(x_hbm, idx_hbm, m_hbm, out_hbm, idx_v, vals_v, m_v, acc_v, sem):
        wid = lax.axis_index("s") * num_cores + lax.axis_index("c")
        base = wid * bpw
        acc = jnp.zeros((num_lanes,), jnp.float32)
        for c in range(nchunks):
            off = base + c * chunk
            pltpu.sync_copy(idx_hbm.at[pl.ds(off, chunk)], idx_v)
            pltpu.sync_copy(m_hbm.at[pl.ds(off, chunk)], m_v)
            pltpu.async_copy(x_hbm.at[idx_v], vals_v, sem).wait()
            for j in range(chunk // num_lanes):
                sl = pl.ds(j * num_lanes, num_lanes)
                acc = acc + vals_v[sl] * m_v[sl]
        acc_v[...] = acc
        pltpu.sync_copy(acc_v, out_hbm.at[wid])

    return k(x_flat, flat_idx, m)


def kernel(input, target, mask):
    b, t, v = input.shape
    n = b * t
    x = input.reshape(n, v)
    m = mask.reshape(n).astype(jnp.float32)
    tgt = target.reshape(n).astype(jnp.int32)

    xt = input.reshape(n // 8, 8, v // 128, 128)
    xt = xt.transpose(0, 2, 1, 3).reshape(-1)
    r = jnp.arange(n, dtype=jnp.int32)
    flat_idx = ((r // 8) * (8 * v) + (tgt // 128) * 1024
                + (r % 8) * 128 + (tgt % 128))

    tc_out = _tc_masked_rowsum(x, m)
    sc_partials = _sc_masked_gather_partials(xt, flat_idx, m)

    a = jnp.sum(tc_out[:, 0, 0])
    msum = jnp.sum(tc_out[:, 0, 1])
    tdot = jnp.sum(sc_partials)

    eps = _SMOOTHING / (v - 1)
    const = (v - 1) * eps * math.log(eps) + _CONFIDENCE * math.log(_CONFIDENCE)
    loss = (const * msum - eps * a - (_CONFIDENCE - eps) * tdot) / msum
    return loss


# speedup vs baseline: 1.0030x; 1.0030x over previous
import functools
import math

import jax
import jax.numpy as jnp
from jax import lax
from jax.experimental import pallas as pl
from jax.experimental.pallas import tpu as pltpu
from jax.experimental.pallas import tpu_sc as plsc

_SMOOTHING = 0.1
_CONFIDENCE = 1.0 - _SMOOTHING

_ROW_BLOCK = 256


def _tc_rowsum_body(x_ref, m_ref, out_ref):
    rs = jnp.sum(x_ref[...], axis=1)
    m = m_ref[...]
    out_ref[0, 0, 0] = jnp.sum(rs * m)
    out_ref[0, 0, 1] = jnp.sum(m)


def _tc_masked_rowsum(x, m):
    n, v = x.shape
    grid = n // _ROW_BLOCK
    return pl.pallas_call(
        _tc_rowsum_body,
        grid=(grid,),
        in_specs=[
            pl.BlockSpec((_ROW_BLOCK, v), lambda i: (i, 0)),
            pl.BlockSpec((_ROW_BLOCK,), lambda i: (i,)),
        ],
        out_specs=pl.BlockSpec(
            (1, 1, 2), lambda i: (i, 0, 0), memory_space=pltpu.SMEM),
        out_shape=jax.ShapeDtypeStruct((grid, 1, 2), jnp.float32),
        compiler_params=pltpu.CompilerParams(
            dimension_semantics=("parallel",)),
    )(x, m)


def _sc_masked_gather_partials(x_flat, flat_idx, m):
    n = flat_idx.shape[0]
    info = plsc.get_sparse_core_info()
    num_cores, num_subcores, num_lanes = (
        info.num_cores, info.num_subcores, info.num_lanes)
    nw = num_cores * num_subcores
    bpw = n // nw
    chunk = 128
    nchunks = bpw // chunk
    mesh = plsc.VectorSubcoreMesh(core_axis_name="c", subcore_axis_name="s")

    @functools.partial(
        pl.kernel,
        mesh=mesh,
        out_type=jax.ShapeDtypeStruct((nw, num_lanes), jnp.float32),
        scratch_types=[
            pltpu.VMEM((chunk,), jnp.int32),
            pltpu.VMEM((chunk,), jnp.float32),
            pltpu.VMEM((chunk,), jnp.float32),
            pltpu.VMEM((num_lanes,), jnp.float32),
            pltpu.SemaphoreType.DMA,
        ],
    )
    def k(x_hbm, idx_hbm, m_hbm, out_hbm, idx_v, vals_v, m_v, acc_v, sem):
        wid = lax.axis_index("s") * num_cores + lax.axis_index("c")
        base = wid * bpw
        acc = jnp.zeros((num_lanes,), jnp.float32)
        for c in range(nchunks):
            off = base + c * chunk
            pltpu.sync_copy(idx_hbm.at[pl.ds(off, chunk)], idx_v)
            pltpu.sync_copy(m_hbm.at[pl.ds(off, chunk)], m_v)
            pltpu.async_copy(x_hbm.at[idx_v], vals_v, sem).wait()
            for j in range(chunk // num_lanes):
                sl = pl.ds(j * num_lanes, num_lanes)
                acc = acc + vals_v[sl] * m_v[sl]
        acc_v[...] = acc
        pltpu.sync_copy(acc_v, out_hbm.at[wid])

    return k(x_flat, flat_idx, m)


def kernel(input, target, mask):
    b, t, v = input.shape
    n = b * t
    x = input.reshape(n, v)
    m = mask.reshape(n).astype(jnp.float32)
    tgt = target.reshape(n).astype(jnp.int32)

    xt = input.reshape(n // 8, 8, v // 128, 128)
    xt = xt.transpose(0, 2, 1, 3).reshape(-1)
    r = jnp.arange(n, dtype=jnp.int32)
    flat_idx = ((r // 8) * (8 * v) + (tgt // 128) * 1024
                + (r % 8) * 128 + (tgt % 128))

    tc_out = _tc_masked_rowsum(x, m)
    sc_partials = _sc_masked_gather_partials(xt, flat_idx, m)

    a = jnp.sum(tc_out[:, 0, 0])
    msum = jnp.sum(tc_out[:, 0, 1])
    tdot = jnp.sum(sc_partials)

    eps = _SMOOTHING / (v - 1)
    const = (v - 1) * eps * math.log(eps) + _CONFIDENCE * math.log(_CONFIDENCE)
    loss = (const * msum - eps * a - (_CONFIDENCE - eps) * tdot) / msum
    return loss


# speedup vs baseline: 1.0035x; 1.0005x over previous
import functools
import math

import jax
import jax.numpy as jnp
from jax import lax
from jax.experimental import pallas as pl
from jax.experimental.pallas import tpu as pltpu
from jax.experimental.pallas import tpu_sc as plsc

_SMOOTHING = 0.1
_CONFIDENCE = 1.0 - _SMOOTHING

_ROW_BLOCK = 256


def _tc_rowsum_body(x_ref, m_ref, out_ref):
    rs = jnp.sum(x_ref[...], axis=1)
    m = m_ref[...]
    out_ref[0, 0, 0] = jnp.sum(rs * m)
    out_ref[0, 0, 1] = jnp.sum(m)


def _tc_masked_rowsum(x, m):
    n, v = x.shape
    grid = n // _ROW_BLOCK
    return pl.pallas_call(
        _tc_rowsum_body,
        grid=(grid,),
        in_specs=[
            pl.BlockSpec((_ROW_BLOCK, v), lambda i: (i, 0)),
            pl.BlockSpec((_ROW_BLOCK,), lambda i: (i,)),
        ],
        out_specs=pl.BlockSpec(
            (1, 1, 2), lambda i: (i, 0, 0), memory_space=pltpu.SMEM),
        out_shape=jax.ShapeDtypeStruct((grid, 1, 2), jnp.float32),
        compiler_params=pltpu.CompilerParams(
            dimension_semantics=("parallel",)),
    )(x, m)


def _sc_masked_gather_partials(x_flat, tgt, m, v):
    n = tgt.shape[0]
    info = plsc.get_sparse_core_info()
    num_cores, num_subcores, num_lanes = (
        info.num_cores, info.num_subcores, info.num_lanes)
    nw = num_cores * num_subcores
    bpw = n // nw
    chunk = 128
    nchunks = bpw // chunk
    mesh = plsc.VectorSubcoreMesh(core_axis_name="c", subcore_axis_name="s")

    @functools.partial(
        pl.kernel,
        mesh=mesh,
        out_type=jax.ShapeDtypeStruct((nw, num_lanes), jnp.float32),
        scratch_types=[
            pltpu.VMEM((chunk,), jnp.int32),
            pltpu.VMEM((chunk,), jnp.int32),
            pltpu.VMEM((chunk,), jnp.float32),
            pltpu.VMEM((chunk,), jnp.float32),
            pltpu.VMEM((num_lanes,), jnp.float32),
            pltpu.SemaphoreType.DMA,
        ],
    )
    def k(x_hbm, tgt_hbm, m_hbm, out_hbm, tgt_v, idx_v, vals_v, m_v, acc_v,
          sem):
        wid = lax.axis_index("s") * num_cores + lax.axis_index("c")
        base = wid * bpw
        lanes = lax.iota(jnp.int32, num_lanes)
        acc = jnp.zeros((num_lanes,), jnp.float32)
        for c in range(nchunks):
            off = base + c * chunk
            pltpu.sync_copy(tgt_hbm.at[pl.ds(off, chunk)], tgt_v)
            pltpu.sync_copy(m_hbm.at[pl.ds(off, chunk)], m_v)
            for j in range(chunk // num_lanes):
                sl = pl.ds(j * num_lanes, num_lanes)
                tv = tgt_v[sl]
                rv = jnp.full((num_lanes,), off + j * num_lanes,
                              jnp.int32) + lanes
                idx_v[sl] = ((rv >> 3) * (8 * v) + (tv >> 7) * 1024
                             + (rv & 7) * 128 + (tv & 127))
            pltpu.async_copy(x_hbm.at[idx_v], vals_v, sem).wait()
            for j in range(chunk // num_lanes):
                sl = pl.ds(j * num_lanes, num_lanes)
                acc = acc + vals_v[sl] * m_v[sl]
        acc_v[...] = acc
        pltpu.sync_copy(acc_v, out_hbm.at[wid])

    return k(x_flat, tgt, m)


def kernel(input, target, mask):
    b, t, v = input.shape
    n = b * t
    x = input.reshape(n, v)
    m = mask.reshape(n).astype(jnp.float32)
    tgt = target.reshape(n).astype(jnp.int32)

    xt = input.reshape(n // 8, 8, v // 128, 128)
    xt = xt.transpose(0, 2, 1, 3).reshape(-1)

    tc_out = _tc_masked_rowsum(x, m)
    sc_partials = _sc_masked_gather_partials(xt, tgt, m, v)

    a = jnp.sum(tc_out[:, 0, 0])
    msum = jnp.sum(tc_out[:, 0, 1])
    tdot = jnp.sum(sc_partials)

    eps = _SMOOTHING / (v - 1)
    const = (v - 1) * eps * math.log(eps) + _CONFIDENCE * math.log(_CONFIDENCE)
    loss = (const * msum - eps * a - (_CONFIDENCE - eps) * tdot) / msum
    return loss
